# baseline (speedup 1.0000x reference)
import functools

import jax
import jax.numpy as jnp
import numpy as np
from jax import lax
from jax.experimental import pallas as pl
from jax.experimental.pallas import tpu as pltpu
from jax.experimental.pallas import tpu_sc as plsc

T, N, E, D, DM, H = 4, 10000, 320000, 128, 64, 128
NP = 10240
DUMP = N
U = 128
UPT = 160
CH = 40
UNITS_PAD = UPT * 16
EP = UNITS_PAD * U
NB = 1000

_mesh = plsc.VectorSubcoreMesh(
    core_axis_name="c", subcore_axis_name="s", num_cores=2, num_subcores=16)


@functools.partial(
    pl.kernel,
    out_type=jax.ShapeDtypeStruct((T, NP), jnp.float32),
    mesh=_mesh,
    scratch_types=[
        pltpu.VMEM((UPT, U), jnp.int32),
        pltpu.VMEM((U,), jnp.float32),
        pltpu.VMEM((1280,), jnp.float32),
        pltpu.VMEM_SHARED((2 * NP,), jnp.float32),
        pltpu.SemaphoreType.DMA,
        pltpu.SemaphoreType.DMA,
    ],
)
def _deg_kernel(dstdeg_hbm, deg_out, idx_v, ones_v, fbuf, degsh, sem_a, sem_b):
    c = lax.axis_index("c")
    w = lax.axis_index("s")
    for i in range(U // 16):
        ones_v[pl.ds(i * 16, 16)] = jnp.ones((16,), jnp.float32)

    def _zb(i, _):
        fbuf[pl.ds(i * 16, 16)] = jnp.zeros((16,), jnp.float32)
        return 0
    lax.fori_loop(0, 1280 // 16, _zb, 0)
    pltpu.sync_copy(fbuf, degsh.at[pl.ds(w * 1280, 1280)])
    plsc.subcore_barrier()

    for tt in range(2):
        tg = c * 2 + tt
        pltpu.sync_copy(
            dstdeg_hbm.at[pl.ds(tg * UNITS_PAD + w * UPT, UPT), :], idx_v)

        pltpu.async_copy(ones_v, degsh.at[idx_v.at[0]], sem_a, add=True)

        def _unit(i, _):
            ub = 2 * i + 1
            un = 2 * i + 2
            pltpu.async_copy(ones_v, degsh.at[idx_v.at[ub]], sem_b, add=True)
            pltpu.make_async_copy(ones_v, degsh.at[idx_v.at[0]], sem_a).wait()
            pltpu.async_copy(ones_v, degsh.at[idx_v.at[un]], sem_a, add=True)
            pltpu.make_async_copy(ones_v, degsh.at[idx_v.at[0]], sem_b).wait()
            return 0
        lax.fori_loop(0, UPT // 2 - 1, _unit, 0)
        pltpu.async_copy(ones_v, degsh.at[idx_v.at[UPT - 1]], sem_b, add=True)
        pltpu.make_async_copy(ones_v, degsh.at[idx_v.at[0]], sem_a).wait()
        pltpu.make_async_copy(ones_v, degsh.at[idx_v.at[0]], sem_b).wait()
    plsc.subcore_barrier()

    for tt in range(2):
        tg = c * 2 + tt
        pltpu.sync_copy(degsh.at[pl.ds(tt * NP + w * 640, 640)],
                        fbuf.at[pl.ds(0, 640)])
        pltpu.sync_copy(fbuf.at[pl.ds(0, 640)],
                        deg_out.at[tg, pl.ds(w * 640, 640)])


@functools.partial(
    pl.kernel,
    out_type=jax.ShapeDtypeStruct((T, NP, D), jnp.float32),
    mesh=_mesh,
    scratch_types=[
        pltpu.VMEM((CH, U), jnp.int32),
        pltpu.VMEM((CH, U), jnp.int32),
        pltpu.VMEM((U, D), jnp.float32),
        pltpu.VMEM((U, D), jnp.float32),
        pltpu.VMEM_SHARED((NP, D), jnp.float32),
        pltpu.SemaphoreType.DMA,
        pltpu.SemaphoreType.DMA,
    ],
)
def _agg_kernel(src_hbm, dst_hbm, hs_hbm, zrows_hbm, agg_out, sidx, didx,
                rows_a, rows_b, aggsh, sem_a, sem_b):
    c = lax.axis_index("c")
    w = lax.axis_index("s")

    def _drain(buf, sem):
        pltpu.make_async_copy(hs_hbm.at[pl.ds(0, U), :], buf, sem).wait()

    for tt in range(2):
        tg = c * 2 + tt

        pltpu.sync_copy(zrows_hbm.at[pl.ds(w * 640, 640), :],
                        aggsh.at[pl.ds(w * 640, 640), :])
        plsc.subcore_barrier()

        for h in range(UPT // CH):
            base = tg * UNITS_PAD + w * UPT + h * CH
            pltpu.sync_copy(src_hbm.at[pl.ds(base, CH), :], sidx)
            pltpu.sync_copy(dst_hbm.at[pl.ds(base, CH), :], didx)

            pltpu.async_copy(hs_hbm.at[sidx.at[0]], rows_a, sem_a)

            def _pair(i, _):
                ua = 2 * i
                ub = 2 * i + 1
                un = jnp.minimum(2 * i + 2, CH - 1)
                pltpu.async_copy(hs_hbm.at[sidx.at[ub]], rows_b, sem_b)
                _drain(rows_a, sem_a)
                pltpu.sync_copy(rows_a, aggsh.at[didx.at[ua]], add=True)
                pltpu.async_copy(hs_hbm.at[sidx.at[un]], rows_a, sem_a)
                _drain(rows_b, sem_b)
                pltpu.sync_copy(rows_b, aggsh.at[didx.at[ub]], add=True)
                return 0
            lax.fori_loop(0, CH // 2, _pair, 0)
            _drain(rows_a, sem_a)
        plsc.subcore_barrier()

        pltpu.sync_copy(aggsh.at[pl.ds(w * 640, 640), :],
                        agg_out.at[tg, pl.ds(w * 640, 640), :])


def _hs_body(feat_ref, mp_ref, deg_ref, w1a_ref, w1b_ref, b1_ref, out_ref):
    h = jnp.dot(feat_ref[0], w1a_ref[...], preferred_element_type=jnp.float32)
    h = h + jnp.dot(mp_ref[0], w1b_ref[...], preferred_element_type=jnp.float32)
    h = h + b1_ref[...]
    scale = lax.rsqrt(jnp.maximum(deg_ref[0], 1.0))
    out_ref[0] = h * scale


def _att_body(agg_ref, deg_ref, wq_ref, wk_ref, wv_ref, wc_ref, ws_ref,
              cz_ref, sz_ref):
    hs = []
    for t in range(T):
        scale = lax.rsqrt(jnp.maximum(deg_ref[t], 1.0))
        hs.append(jnp.maximum(agg_ref[t] * scale, 0.0))
    qs = [jnp.dot(h, wq_ref[...], preferred_element_type=jnp.float32)
          for h in hs]
    ks = [jnp.dot(h, wk_ref[...], preferred_element_type=jnp.float32)
          for h in hs]
    vs = [jnp.dot(h, wv_ref[...], preferred_element_type=jnp.float32)
          for h in hs]
    inv = np.float32(1.0 / np.sqrt(H))
    for t in range(T):
        sc = [jnp.sum(qs[t] * ks[s], axis=1, keepdims=True) * inv
              for s in range(T)]
        m = jnp.maximum(jnp.maximum(sc[0], sc[1]), jnp.maximum(sc[2], sc[3]))
        es = [jnp.exp(sc[s] - m) for s in range(T)]
        den = es[0] + es[1] + es[2] + es[3]
        ctx = (es[0] * vs[0] + es[1] * vs[1]
               + es[2] * vs[2] + es[3] * vs[3]) / den
        cz_ref[t] = jnp.dot(ctx, wc_ref[...], preferred_element_type=jnp.float32)
        sz_ref[t] = jnp.dot(ctx, ws_ref[...], preferred_element_type=jnp.float32)


def kernel(features, edge_index, mp_emb, W1, b1, Wq, Wk, Wv, Wc, Ws):
    f32 = jnp.float32
    src = edge_index[:, 0, :]
    dst = edge_index[:, 1, :]
    toff = (jnp.arange(T, dtype=jnp.int32) * NP)[:, None]
    srcp = jnp.pad(src + toff, ((0, 0), (0, EP - E))).reshape(T * UNITS_PAD, U)
    dstp = jnp.pad(dst, ((0, 0), (0, EP - E)),
                   constant_values=DUMP)
    dstdeg = (dstp + ((jnp.arange(T, dtype=jnp.int32) % 2) * NP)[:, None]
              ).reshape(T * UNITS_PAD, U)
    dstp = dstp.reshape(T * UNITS_PAD, U)

    degp = _deg_kernel(dstdeg)
    degp3 = degp[:, :, None]

    W1a, W1b = W1[:D], W1[D:]
    b1_2d = b1[None, :]
    hsp = pl.pallas_call(
        _hs_body,
        grid=(T, N // NB),
        in_specs=[
            pl.BlockSpec((1, NB, D), lambda t, i: (t, i, 0)),
            pl.BlockSpec((1, NB, DM), lambda t, i: (t, i, 0)),
            pl.BlockSpec((1, NB, 1), lambda t, i: (t, i, 0)),
            pl.BlockSpec((D, H), lambda t, i: (0, 0)),
            pl.BlockSpec((DM, H), lambda t, i: (0, 0)),
            pl.BlockSpec((1, H), lambda t, i: (0, 0)),
        ],
        out_specs=pl.BlockSpec((1, NB, H), lambda t, i: (t, i, 0)),
        out_shape=jax.ShapeDtypeStruct((T, NP, H), f32),
    )(features, mp_emb, degp3, W1a, W1b, b1_2d)

    zrows = jnp.zeros((NP, D), f32)
    aggp = _agg_kernel(srcp, dstp, hsp.reshape(T * NP, H), zrows)

    cz, sz = pl.pallas_call(
        _att_body,
        grid=(N // NB,),
        in_specs=[
            pl.BlockSpec((T, NB, H), lambda i: (0, i, 0)),
            pl.BlockSpec((T, NB, 1), lambda i: (0, i, 0)),
            pl.BlockSpec((H, H), lambda i: (0, 0)),
            pl.BlockSpec((H, H), lambda i: (0, 0)),
            pl.BlockSpec((H, H), lambda i: (0, 0)),
            pl.BlockSpec((H, H), lambda i: (0, 0)),
            pl.BlockSpec((H, H), lambda i: (0, 0)),
        ],
        out_specs=[
            pl.BlockSpec((T, NB, H), lambda i: (0, i, 0)),
            pl.BlockSpec((T, NB, H), lambda i: (0, i, 0)),
        ],
        out_shape=[
            jax.ShapeDtypeStruct((T, N, H), f32),
            jax.ShapeDtypeStruct((T, N, H), f32),
        ],
    )(aggp, degp3, Wq, Wk, Wv, Wc, Ws)

    return (cz, sz)


# speedup vs baseline: 1.0036x; 1.0036x over previous
import functools

import jax
import jax.numpy as jnp
import numpy as np
from jax import lax
from jax.experimental import pallas as pl
from jax.experimental.pallas import tpu as pltpu
from jax.experimental.pallas import tpu_sc as plsc

T, N, E, D, DM, H = 4, 10000, 320000, 128, 64, 128
NP = 10240
DUMP = N
U = 128
UPT = 160
CH = 40
UNITS_PAD = UPT * 16
EP = UNITS_PAD * U
NB = 1000

_mesh = plsc.VectorSubcoreMesh(
    core_axis_name="c", subcore_axis_name="s", num_cores=2, num_subcores=16)


@functools.partial(
    pl.kernel,
    out_type=jax.ShapeDtypeStruct((T, NP), jnp.float32),
    mesh=_mesh,
    scratch_types=[
        pltpu.VMEM((UPT, U), jnp.int32),
        pltpu.VMEM((U,), jnp.float32),
        pltpu.VMEM((1280,), jnp.float32),
        pltpu.VMEM_SHARED((2 * NP,), jnp.float32),
        pltpu.SemaphoreType.DMA,
        pltpu.SemaphoreType.DMA,
    ],
)
def _deg_kernel(dstdeg_hbm, deg_out, idx_v, ones_v, fbuf, degsh, sem_a, sem_b):
    c = lax.axis_index("c")
    w = lax.axis_index("s")
    for i in range(U // 16):
        ones_v[pl.ds(i * 16, 16)] = jnp.ones((16,), jnp.float32)

    def _zb(i, _):
        fbuf[pl.ds(i * 16, 16)] = jnp.zeros((16,), jnp.float32)
        return 0
    lax.fori_loop(0, 1280 // 16, _zb, 0)
    pltpu.sync_copy(fbuf, degsh.at[pl.ds(w * 1280, 1280)])
    plsc.subcore_barrier()

    for tt in range(2):
        tg = c * 2 + tt
        pltpu.sync_copy(
            dstdeg_hbm.at[pl.ds(tg * UNITS_PAD + w * UPT, UPT), :], idx_v)

        pltpu.async_copy(ones_v, degsh.at[idx_v.at[0]], sem_a, add=True)

        def _unit(i, _):
            ub = 2 * i + 1
            un = 2 * i + 2
            pltpu.async_copy(ones_v, degsh.at[idx_v.at[ub]], sem_b, add=True)
            pltpu.make_async_copy(ones_v, degsh.at[idx_v.at[0]], sem_a).wait()
            pltpu.async_copy(ones_v, degsh.at[idx_v.at[un]], sem_a, add=True)
            pltpu.make_async_copy(ones_v, degsh.at[idx_v.at[0]], sem_b).wait()
            return 0
        lax.fori_loop(0, UPT // 2 - 1, _unit, 0)
        pltpu.async_copy(ones_v, degsh.at[idx_v.at[UPT - 1]], sem_b, add=True)
        pltpu.make_async_copy(ones_v, degsh.at[idx_v.at[0]], sem_a).wait()
        pltpu.make_async_copy(ones_v, degsh.at[idx_v.at[0]], sem_b).wait()
    plsc.subcore_barrier()

    for tt in range(2):
        tg = c * 2 + tt
        pltpu.sync_copy(degsh.at[pl.ds(tt * NP + w * 640, 640)],
                        fbuf.at[pl.ds(0, 640)])
        pltpu.sync_copy(fbuf.at[pl.ds(0, 640)],
                        deg_out.at[tg, pl.ds(w * 640, 640)])


@functools.partial(
    pl.kernel,
    out_type=jax.ShapeDtypeStruct((T, NP, D), jnp.float32),
    mesh=_mesh,
    scratch_types=[
        pltpu.VMEM((CH, U), jnp.int32),
        pltpu.VMEM((CH, U), jnp.int32),
        pltpu.VMEM((U, D), jnp.float32),
        pltpu.VMEM((U, D), jnp.float32),
        pltpu.VMEM_SHARED((NP, D), jnp.float32),
        pltpu.SemaphoreType.DMA,
        pltpu.SemaphoreType.DMA,
    ],
)
def _agg_kernel(src_hbm, dst_hbm, hs_hbm, agg_out, sidx, didx,
                rows_a, rows_b, aggsh, sem_a, sem_b):
    c = lax.axis_index("c")
    w = lax.axis_index("s")

    def _drain(buf, sem):
        pltpu.make_async_copy(hs_hbm.at[pl.ds(0, U), :], buf, sem).wait()

    for tt in range(2):
        tg = c * 2 + tt

        def _zr(i, _):
            for k in range(D // 16):
                rows_a[i, pl.ds(k * 16, 16)] = jnp.zeros((16,), jnp.float32)
            return 0
        lax.fori_loop(0, U, _zr, 0)
        for r in range(640 // U):
            pltpu.sync_copy(rows_a, aggsh.at[pl.ds(w * 640 + r * U, U), :])
        plsc.subcore_barrier()

        for h in range(UPT // CH):
            base = tg * UNITS_PAD + w * UPT + h * CH
            pltpu.sync_copy(src_hbm.at[pl.ds(base, CH), :], sidx)
            pltpu.sync_copy(dst_hbm.at[pl.ds(base, CH), :], didx)

            pltpu.async_copy(hs_hbm.at[sidx.at[0]], rows_a, sem_a)

            def _pair(i, _):
                ua = 2 * i
                ub = 2 * i + 1
                un = jnp.minimum(2 * i + 2, CH - 1)
                pltpu.async_copy(hs_hbm.at[sidx.at[ub]], rows_b, sem_b)
                _drain(rows_a, sem_a)
                pltpu.sync_copy(rows_a, aggsh.at[didx.at[ua]], add=True)
                pltpu.async_copy(hs_hbm.at[sidx.at[un]], rows_a, sem_a)
                _drain(rows_b, sem_b)
                pltpu.sync_copy(rows_b, aggsh.at[didx.at[ub]], add=True)
                return 0
            lax.fori_loop(0, CH // 2, _pair, 0)
            _drain(rows_a, sem_a)
        plsc.subcore_barrier()

        for r in range(640 // U):
            pltpu.sync_copy(aggsh.at[pl.ds(w * 640 + r * U, U), :], rows_a)
            pltpu.sync_copy(rows_a,
                            agg_out.at[tg, pl.ds(w * 640 + r * U, U), :])


def _hs_body(feat_ref, mp_ref, deg_ref, w1a_ref, w1b_ref, b1_ref, out_ref):
    h = jnp.dot(feat_ref[0], w1a_ref[...], preferred_element_type=jnp.float32)
    h = h + jnp.dot(mp_ref[0], w1b_ref[...], preferred_element_type=jnp.float32)
    h = h + b1_ref[...]
    scale = lax.rsqrt(jnp.maximum(deg_ref[0], 1.0))
    out_ref[0] = h * scale


def _att_body(agg_ref, deg_ref, wq_ref, wk_ref, wv_ref, wc_ref, ws_ref,
              cz_ref, sz_ref):
    hs = []
    for t in range(T):
        scale = lax.rsqrt(jnp.maximum(deg_ref[t], 1.0))
        hs.append(jnp.maximum(agg_ref[t] * scale, 0.0))
    qs = [jnp.dot(h, wq_ref[...], preferred_element_type=jnp.float32)
          for h in hs]
    ks = [jnp.dot(h, wk_ref[...], preferred_element_type=jnp.float32)
          for h in hs]
    vs = [jnp.dot(h, wv_ref[...], preferred_element_type=jnp.float32)
          for h in hs]
    inv = np.float32(1.0 / np.sqrt(H))
    for t in range(T):
        sc = [jnp.sum(qs[t] * ks[s], axis=1, keepdims=True) * inv
              for s in range(T)]
        m = jnp.maximum(jnp.maximum(sc[0], sc[1]), jnp.maximum(sc[2], sc[3]))
        es = [jnp.exp(sc[s] - m) for s in range(T)]
        den = es[0] + es[1] + es[2] + es[3]
        ctx = (es[0] * vs[0] + es[1] * vs[1]
               + es[2] * vs[2] + es[3] * vs[3]) / den
        cz_ref[t] = jnp.dot(ctx, wc_ref[...], preferred_element_type=jnp.float32)
        sz_ref[t] = jnp.dot(ctx, ws_ref[...], preferred_element_type=jnp.float32)


def kernel(features, edge_index, mp_emb, W1, b1, Wq, Wk, Wv, Wc, Ws):
    f32 = jnp.float32
    src = edge_index[:, 0, :]
    dst = edge_index[:, 1, :]
    toff = (jnp.arange(T, dtype=jnp.int32) * NP)[:, None]
    srcp = jnp.pad(src + toff, ((0, 0), (0, EP - E))).reshape(T * UNITS_PAD, U)
    dstp = jnp.pad(dst, ((0, 0), (0, EP - E)),
                   constant_values=DUMP)
    dstdeg = (dstp + ((jnp.arange(T, dtype=jnp.int32) % 2) * NP)[:, None]
              ).reshape(T * UNITS_PAD, U)
    dstp = dstp.reshape(T * UNITS_PAD, U)

    degp = _deg_kernel(dstdeg)
    degp3 = degp[:, :, None]

    W1a, W1b = W1[:D], W1[D:]
    b1_2d = b1[None, :]
    hsp = pl.pallas_call(
        _hs_body,
        grid=(T, N // NB),
        in_specs=[
            pl.BlockSpec((1, NB, D), lambda t, i: (t, i, 0)),
            pl.BlockSpec((1, NB, DM), lambda t, i: (t, i, 0)),
            pl.BlockSpec((1, NB, 1), lambda t, i: (t, i, 0)),
            pl.BlockSpec((D, H), lambda t, i: (0, 0)),
            pl.BlockSpec((DM, H), lambda t, i: (0, 0)),
            pl.BlockSpec((1, H), lambda t, i: (0, 0)),
        ],
        out_specs=pl.BlockSpec((1, NB, H), lambda t, i: (t, i, 0)),
        out_shape=jax.ShapeDtypeStruct((T, NP, H), f32),
    )(features, mp_emb, degp3, W1a, W1b, b1_2d)

    aggp = _agg_kernel(srcp, dstp, hsp.reshape(T * NP, H))

    cz, sz = pl.pallas_call(
        _att_body,
        grid=(N // NB,),
        in_specs=[
            pl.BlockSpec((T, NB, H), lambda i: (0, i, 0)),
            pl.BlockSpec((T, NB, 1), lambda i: (0, i, 0)),
            pl.BlockSpec((H, H), lambda i: (0, 0)),
            pl.BlockSpec((H, H), lambda i: (0, 0)),
            pl.BlockSpec((H, H), lambda i: (0, 0)),
            pl.BlockSpec((H, H), lambda i: (0, 0)),
            pl.BlockSpec((H, H), lambda i: (0, 0)),
        ],
        out_specs=[
            pl.BlockSpec((T, NB, H), lambda i: (0, i, 0)),
            pl.BlockSpec((T, NB, H), lambda i: (0, i, 0)),
        ],
        out_shape=[
            jax.ShapeDtypeStruct((T, N, H), f32),
            jax.ShapeDtypeStruct((T, N, H), f32),
        ],
    )(aggp, degp3, Wq, Wk, Wv, Wc, Ws)

    return (cz, sz)


# speedup vs baseline: 1.0335x; 1.0298x over previous
import functools

import jax
import jax.numpy as jnp
import numpy as np
from jax import lax
from jax.experimental import pallas as pl
from jax.experimental.pallas import tpu as pltpu
from jax.experimental.pallas import tpu_sc as plsc

T, N, E, D, DM, H = 4, 10000, 320000, 128, 64, 128
NP = 10240
DUMP = N
U = 128
UPT = 160
CH = 16
UNITS_PAD = UPT * 16
EP = UNITS_PAD * U
NB = 1000

_mesh = plsc.VectorSubcoreMesh(
    core_axis_name="c", subcore_axis_name="s", num_cores=2, num_subcores=16)


@functools.partial(
    pl.kernel,
    out_type=jax.ShapeDtypeStruct((T, NP), jnp.float32),
    mesh=_mesh,
    scratch_types=[
        pltpu.VMEM((UPT, U), jnp.int32),
        pltpu.VMEM((U,), jnp.float32),
        pltpu.VMEM((1280,), jnp.float32),
        pltpu.VMEM_SHARED((2 * NP,), jnp.float32),
        pltpu.SemaphoreType.DMA,
        pltpu.SemaphoreType.DMA,
    ],
)
def _deg_kernel(dstdeg_hbm, deg_out, idx_v, ones_v, fbuf, degsh, sem_a, sem_b):
    c = lax.axis_index("c")
    w = lax.axis_index("s")
    for i in range(U // 16):
        ones_v[pl.ds(i * 16, 16)] = jnp.ones((16,), jnp.float32)

    def _zb(i, _):
        fbuf[pl.ds(i * 16, 16)] = jnp.zeros((16,), jnp.float32)
        return 0
    lax.fori_loop(0, 1280 // 16, _zb, 0)
    pltpu.sync_copy(fbuf, degsh.at[pl.ds(w * 1280, 1280)])
    plsc.subcore_barrier()

    for tt in range(2):
        tg = c * 2 + tt
        pltpu.sync_copy(
            dstdeg_hbm.at[pl.ds(tg * UNITS_PAD + w * UPT, UPT), :], idx_v)

        pltpu.async_copy(ones_v, degsh.at[idx_v.at[0]], sem_a, add=True)

        def _unit(i, _):
            ub = 2 * i + 1
            un = 2 * i + 2
            pltpu.async_copy(ones_v, degsh.at[idx_v.at[ub]], sem_b, add=True)
            pltpu.make_async_copy(ones_v, degsh.at[idx_v.at[0]], sem_a).wait()
            pltpu.async_copy(ones_v, degsh.at[idx_v.at[un]], sem_a, add=True)
            pltpu.make_async_copy(ones_v, degsh.at[idx_v.at[0]], sem_b).wait()
            return 0
        lax.fori_loop(0, UPT // 2 - 1, _unit, 0)
        pltpu.async_copy(ones_v, degsh.at[idx_v.at[UPT - 1]], sem_b, add=True)
        pltpu.make_async_copy(ones_v, degsh.at[idx_v.at[0]], sem_a).wait()
        pltpu.make_async_copy(ones_v, degsh.at[idx_v.at[0]], sem_b).wait()
    plsc.subcore_barrier()

    for tt in range(2):
        tg = c * 2 + tt
        pltpu.sync_copy(degsh.at[pl.ds(tt * NP + w * 640, 640)],
                        fbuf.at[pl.ds(0, 640)])
        pltpu.sync_copy(fbuf.at[pl.ds(0, 640)],
                        deg_out.at[tg, pl.ds(w * 640, 640)])


@functools.partial(
    pl.kernel,
    out_type=jax.ShapeDtypeStruct((T, NP, D), jnp.float32),
    mesh=_mesh,
    scratch_types=[
        pltpu.VMEM((CH, U), jnp.int32),
        pltpu.VMEM((CH, U), jnp.int32),
        pltpu.VMEM((CH, U), jnp.int32),
        pltpu.VMEM((CH, U), jnp.int32),
        pltpu.VMEM((U, D), jnp.float32),
        pltpu.VMEM((U, D), jnp.float32),
        pltpu.VMEM_SHARED((NP, D), jnp.float32),
        pltpu.SemaphoreType.DMA,
        pltpu.SemaphoreType.DMA,
        pltpu.SemaphoreType.DMA,
    ],
)
def _agg_kernel(src_hbm, dst_hbm, hs_hbm, agg_out, sidx_a, sidx_b,
                didx_a, didx_b, rows_a, rows_b, aggsh, sem_a, sem_b, sem_i):
    c = lax.axis_index("c")
    w = lax.axis_index("s")
    nch = UPT // CH

    def _drain(buf, sem):
        pltpu.make_async_copy(hs_hbm.at[pl.ds(0, U), :], buf, sem).wait()

    for tt in range(2):
        tg = c * 2 + tt

        def _zr(i, _):
            for k in range(D // 16):
                rows_a[i, pl.ds(k * 16, 16)] = jnp.zeros((16,), jnp.float32)
            return 0
        lax.fori_loop(0, U, _zr, 0)
        for r in range(640 // U):
            pltpu.sync_copy(rows_a, aggsh.at[pl.ds(w * 640 + r * U, U), :])
        plsc.subcore_barrier()

        base0 = tg * UNITS_PAD + w * UPT
        pltpu.sync_copy(src_hbm.at[pl.ds(base0, CH), :], sidx_a)
        pltpu.sync_copy(dst_hbm.at[pl.ds(base0, CH), :], didx_a)
        pltpu.async_copy(hs_hbm.at[sidx_a.at[0]], rows_a, sem_a)
        for h in range(nch):
            s_cur, d_cur = (sidx_a, didx_a) if h % 2 == 0 else (sidx_b, didx_b)
            s_nxt, d_nxt = (sidx_b, didx_b) if h % 2 == 0 else (sidx_a, didx_a)
            last = h == nch - 1
            if not last:
                base_n = base0 + (h + 1) * CH
                pltpu.async_copy(src_hbm.at[pl.ds(base_n, CH), :], s_nxt,
                                 sem_i)
                pltpu.async_copy(dst_hbm.at[pl.ds(base_n, CH), :], d_nxt,
                                 sem_i)

            def _pair(i, _, s_cur=s_cur, d_cur=d_cur):
                ub = 2 * i + 1
                un = 2 * i + 2
                pltpu.async_copy(hs_hbm.at[s_cur.at[ub]], rows_b, sem_b)
                _drain(rows_a, sem_a)
                pltpu.sync_copy(rows_a, aggsh.at[d_cur.at[2 * i]], add=True)
                pltpu.async_copy(hs_hbm.at[s_cur.at[un]], rows_a, sem_a)
                _drain(rows_b, sem_b)
                pltpu.sync_copy(rows_b, aggsh.at[d_cur.at[ub]], add=True)
                return 0
            lax.fori_loop(0, CH // 2 - 1, _pair, 0)

            pltpu.async_copy(hs_hbm.at[s_cur.at[CH - 1]], rows_b, sem_b)
            _drain(rows_a, sem_a)
            pltpu.sync_copy(rows_a, aggsh.at[d_cur.at[CH - 2]], add=True)
            if not last:
                pltpu.make_async_copy(src_hbm.at[pl.ds(base0, CH), :],
                                      s_nxt, sem_i).wait()
                pltpu.make_async_copy(dst_hbm.at[pl.ds(base0, CH), :],
                                      d_nxt, sem_i).wait()
                pltpu.async_copy(hs_hbm.at[s_nxt.at[0]], rows_a, sem_a)
            _drain(rows_b, sem_b)
            pltpu.sync_copy(rows_b, aggsh.at[d_cur.at[CH - 1]], add=True)
        plsc.subcore_barrier()

        for r in range(640 // U):
            pltpu.sync_copy(aggsh.at[pl.ds(w * 640 + r * U, U), :], rows_a)
            pltpu.sync_copy(rows_a,
                            agg_out.at[tg, pl.ds(w * 640 + r * U, U), :])


def _hs_body(feat_ref, mp_ref, deg_ref, w1a_ref, w1b_ref, b1_ref, out_ref):
    h = jnp.dot(feat_ref[0], w1a_ref[...], preferred_element_type=jnp.float32)
    h = h + jnp.dot(mp_ref[0], w1b_ref[...], preferred_element_type=jnp.float32)
    h = h + b1_ref[...]
    scale = lax.rsqrt(jnp.maximum(deg_ref[0], 1.0))
    out_ref[0] = h * scale


def _att_body(agg_ref, deg_ref, wq_ref, wk_ref, wv_ref, wc_ref, ws_ref,
              cz_ref, sz_ref):
    hs = []
    for t in range(T):
        scale = lax.rsqrt(jnp.maximum(deg_ref[t], 1.0))
        hs.append(jnp.maximum(agg_ref[t] * scale, 0.0))
    qs = [jnp.dot(h, wq_ref[...], preferred_element_type=jnp.float32)
          for h in hs]
    ks = [jnp.dot(h, wk_ref[...], preferred_element_type=jnp.float32)
          for h in hs]
    vs = [jnp.dot(h, wv_ref[...], preferred_element_type=jnp.float32)
          for h in hs]
    inv = np.float32(1.0 / np.sqrt(H))
    for t in range(T):
        sc = [jnp.sum(qs[t] * ks[s], axis=1, keepdims=True) * inv
              for s in range(T)]
        m = jnp.maximum(jnp.maximum(sc[0], sc[1]), jnp.maximum(sc[2], sc[3]))
        es = [jnp.exp(sc[s] - m) for s in range(T)]
        den = es[0] + es[1] + es[2] + es[3]
        ctx = (es[0] * vs[0] + es[1] * vs[1]
               + es[2] * vs[2] + es[3] * vs[3]) / den
        cz_ref[t] = jnp.dot(ctx, wc_ref[...], preferred_element_type=jnp.float32)
        sz_ref[t] = jnp.dot(ctx, ws_ref[...], preferred_element_type=jnp.float32)


def kernel(features, edge_index, mp_emb, W1, b1, Wq, Wk, Wv, Wc, Ws):
    f32 = jnp.float32
    src = edge_index[:, 0, :]
    dst = edge_index[:, 1, :]
    toff = (jnp.arange(T, dtype=jnp.int32) * NP)[:, None]
    srcp = jnp.pad(src + toff, ((0, 0), (0, EP - E))).reshape(T * UNITS_PAD, U)
    dstp = jnp.pad(dst, ((0, 0), (0, EP - E)),
                   constant_values=DUMP)
    dstdeg = (dstp + ((jnp.arange(T, dtype=jnp.int32) % 2) * NP)[:, None]
              ).reshape(T * UNITS_PAD, U)
    dstp = dstp.reshape(T * UNITS_PAD, U)

    degp = _deg_kernel(dstdeg)
    degp3 = degp[:, :, None]

    W1a, W1b = W1[:D], W1[D:]
    b1_2d = b1[None, :]
    hsp = pl.pallas_call(
        _hs_body,
        grid=(T, N // NB),
        in_specs=[
            pl.BlockSpec((1, NB, D), lambda t, i: (t, i, 0)),
            pl.BlockSpec((1, NB, DM), lambda t, i: (t, i, 0)),
            pl.BlockSpec((1, NB, 1), lambda t, i: (t, i, 0)),
            pl.BlockSpec((D, H), lambda t, i: (0, 0)),
            pl.BlockSpec((DM, H), lambda t, i: (0, 0)),
            pl.BlockSpec((1, H), lambda t, i: (0, 0)),
        ],
        out_specs=pl.BlockSpec((1, NB, H), lambda t, i: (t, i, 0)),
        out_shape=jax.ShapeDtypeStruct((T, NP, H), f32),
    )(features, mp_emb, degp3, W1a, W1b, b1_2d)

    aggp = _agg_kernel(srcp, dstp, hsp.reshape(T * NP, H))

    cz, sz = pl.pallas_call(
        _att_body,
        grid=(N // NB,),
        in_specs=[
            pl.BlockSpec((T, NB, H), lambda i: (0, i, 0)),
            pl.BlockSpec((T, NB, 1), lambda i: (0, i, 0)),
            pl.BlockSpec((H, H), lambda i: (0, 0)),
            pl.BlockSpec((H, H), lambda i: (0, 0)),
            pl.BlockSpec((H, H), lambda i: (0, 0)),
            pl.BlockSpec((H, H), lambda i: (0, 0)),
            pl.BlockSpec((H, H), lambda i: (0, 0)),
        ],
        out_specs=[
            pl.BlockSpec((T, NB, H), lambda i: (0, i, 0)),
            pl.BlockSpec((T, NB, H), lambda i: (0, i, 0)),
        ],
        out_shape=[
            jax.ShapeDtypeStruct((T, N, H), f32),
            jax.ShapeDtypeStruct((T, N, H), f32),
        ],
    )(aggp, degp3, Wq, Wk, Wv, Wc, Ws)

    return (cz, sz)
